# baseline (speedup 1.0000x reference)
import functools

import numpy as np
import jax
import jax.numpy as jnp
from jax.experimental import pallas as pl
from jax.experimental.pallas import tpu as pltpu

_N = 32
_OH1 = 28
_P1 = 14
_OH2 = 10
_D = 5
_F = 16 * _D * _D
_CPAD = 128
_B_TILE = 512
_K1 = 5 * _N


def _conv1_sel_np(parity):
    s = np.zeros((5, 5, _K1, _P1), np.float32)
    for j in range(5):
        for kw in range(5):
            for pw in range(_P1):
                s[j, kw, j * _N + 2 * pw + parity + kw, pw] = 1.0
    return s


def _conv2_sel_np():
    s = np.zeros((25, _P1 * _P1, _OH2 * _OH2), np.float32)
    for kh in range(5):
        for kw in range(5):
            for ph in range(_OH2):
                for pw in range(_OH2):
                    s[kh * 5 + kw, (ph + kh) * _P1 + (pw + kw),
                      ph * _OH2 + pw] = 1.0
    return s.reshape(25, _P1, _P1, _OH2 * _OH2)


def _pool2_mat_np():
    m = np.zeros((16 * _OH2 * _OH2, _F), np.float32)
    for c in range(16):
        for oh in range(_D):
            for ow in range(_D):
                col = c * _D * _D + oh * _D + ow
                for dh in range(2):
                    for dw in range(2):
                        m[c * 100 + (2 * oh + dh) * _OH2 + (2 * ow + dw),
                          col] = 0.25
    return m


_CONV1_SEL_E = _conv1_sel_np(0)
_CONV1_SEL_O = _conv1_sel_np(1)
_CONV2_SEL = _conv2_sel_np()
_POOL2 = _pool2_mat_np()


def _lenet_body(x_ref, w1st_ref, b1_ref, c2_ref, b2_ref, p2_ref,
                wf1_ref, wf2_ref, wf3_ref, bfc_ref,
                o_ref, *, b_tile):
    f32 = jnp.float32
    bf16 = jnp.bfloat16

    xf = x_ref[...]
    xh = xf.astype(bf16)
    r1 = xf - xh.astype(f32)
    xm = r1.astype(bf16)
    xl = (r1 - xm.astype(f32)).astype(bf16)

    rows = []
    for r in range(_OH1):
        hs = [xh[:, (r + j) * _N:(r + j + 1) * _N] for j in range(5)]
        ms = [xm[:, (r + j) * _N:(r + j + 1) * _N] for j in range(5)]
        ls = [xl[:, (r + j) * _N:(r + j + 1) * _N] for j in range(5)]
        rows.append(jnp.concatenate(hs + ms + ls + hs + ms + hs, axis=1))
    xb = jnp.concatenate(rows, axis=0)

    b1 = b1_ref[...]
    ze = jnp.dot(xb, w1st_ref[:, :6 * _P1], preferred_element_type=f32) + b1
    zo = jnp.dot(xb, w1st_ref[:, 6 * _P1:], preferred_element_type=f32) + b1
    ae = jnp.tanh(ze).astype(bf16).astype(f32).reshape(_OH1, b_tile, 6 * _P1)
    ao = jnp.tanh(zo).astype(bf16).astype(f32).reshape(_OH1, b_tile, 6 * _P1)

    ht = jnp.concatenate(
        [0.25 * (ae[2 * p] + ae[2 * p + 1] + ao[2 * p] + ao[2 * p + 1])
         for p in range(_P1)], axis=1)
    z2 = jnp.dot(ht.astype(bf16), c2_ref[...],
                 preferred_element_type=f32) + b2_ref[...]
    a2 = jnp.tanh(z2)

    feats = jnp.dot(a2.astype(bf16), p2_ref[...],
                    preferred_element_type=f32)

    h = jnp.tanh(jnp.dot(feats.astype(bf16), wf1_ref[...],
                         preferred_element_type=f32) + bfc_ref[:, :120])
    h = jnp.tanh(jnp.dot(h.astype(bf16), wf2_ref[...],
                         preferred_element_type=f32) + bfc_ref[:, 120:204])
    logits = jnp.dot(h.astype(bf16), wf3_ref[...],
                     preferred_element_type=f32) + bfc_ref[:, 204:]
    logits = logits - jnp.max(logits, axis=-1, keepdims=True)
    e = jnp.exp(logits)
    o_ref[...] = e * pl.reciprocal(jnp.sum(e, axis=-1, keepdims=True),
                                   approx=True)


def kernel(x, w_conv1, b_conv1, w_conv2, b_conv2, w_fc1, b_fc1,
           w_fc2, b_fc2, w_fc3, b_fc3, mean, std):
    f32 = jnp.float32
    bf16 = jnp.bfloat16
    B = x.shape[0]
    C = w_fc3.shape[0]
    b_tile = _B_TILE

    w1 = w_conv1.reshape(6, 5, 5).astype(f32)
    w1s = w1 / std
    b1s = b_conv1.astype(f32) - (mean / std) * jnp.sum(w1, axis=(1, 2))
    w1h = w1s.astype(bf16).astype(f32)
    w1lr = w1s - w1h
    w1l = w1lr.astype(bf16).astype(f32)
    w1ll = w1lr - w1l
    sele = jnp.asarray(_CONV1_SEL_E)
    selo = jnp.asarray(_CONV1_SEL_O)

    def embed(w):
        return jnp.concatenate(
            [jnp.einsum("cjk,jkrw->rcw", w, sele).reshape(_K1, 6 * _P1),
             jnp.einsum("cjk,jkrw->rcw", w, selo).reshape(_K1, 6 * _P1)],
            axis=1)

    w1h_m = embed(w1h)
    w1l_m = embed(w1l)
    w1ll_m = embed(w1ll)
    w1st = jnp.concatenate(
        [w1h_m, w1h_m, w1h_m, w1l_m, w1l_m, w1ll_m],
        axis=0).astype(bf16)
    b1row = jnp.repeat(b1s, _P1).reshape(1, 6 * _P1)

    w2r = jnp.transpose(w_conv2.astype(f32), (1, 2, 3, 0)).reshape(6, 25, 16)
    c2big = jnp.einsum("cao,ahwp->hcwop", w2r, jnp.asarray(_CONV2_SEL))
    c2big = c2big.reshape(_P1 * 6 * _P1, 16 * 100).astype(bf16)
    b2row = jnp.repeat(b_conv2.astype(f32), _OH2 * _OH2).reshape(1, 1600)

    pool2 = jnp.asarray(_POOL2, bf16)

    wf1 = w_fc1.T.astype(bf16)
    wf2 = w_fc2.T.astype(bf16)
    wf3p = jnp.zeros((84, _CPAD), f32).at[:, :C].set(
        w_fc3.T.astype(f32)).astype(bf16)
    bf3p = jnp.full((_CPAD,), -1e30, f32).at[:C].set(b_fc3.astype(f32))
    bias_fc = jnp.concatenate(
        [b_fc1.astype(f32), b_fc2.astype(f32), bf3p]).reshape(1, 204 + _CPAD)

    Bp = ((B + b_tile - 1) // b_tile) * b_tile
    x2d = x.reshape(B, _N * _N).astype(f32)
    if Bp != B:
        x2d = jnp.pad(x2d, ((0, Bp - B), (0, 0)))

    weights = (w1st, b1row, c2big, b2row, pool2, wf1, wf2, wf3p, bias_fc)

    def run(xs, w1st, b1row, c2big, b2row, pool2, wf1, wf2, wf3p,
            bias_fc):
        bs = xs.shape[0]
        kern = functools.partial(_lenet_body, b_tile=b_tile)
        const = lambda i: (0, 0)
        return pl.pallas_call(
            kern,
            out_shape=jax.ShapeDtypeStruct((bs, _CPAD), f32),
            grid=(bs // b_tile,),
            in_specs=[
                pl.BlockSpec((b_tile, _N * _N), lambda i: (i, 0)),
                pl.BlockSpec(w1st.shape, const),
                pl.BlockSpec(b1row.shape, const),
                pl.BlockSpec(c2big.shape, const),
                pl.BlockSpec(b2row.shape, const),
                pl.BlockSpec(pool2.shape, const),
                pl.BlockSpec(wf1.shape, const),
                pl.BlockSpec(wf2.shape, const),
                pl.BlockSpec(wf3p.shape, const),
                pl.BlockSpec(bias_fc.shape, const),
            ],
            out_specs=pl.BlockSpec((b_tile, _CPAD), lambda i: (i, 0)),
            compiler_params=pltpu.CompilerParams(
                dimension_semantics=("parallel",)),
        )(xs, w1st, b1row, c2big, b2row, pool2, wf1, wf2, wf3p, bias_fc)

    out = run(x2d, *weights)
    return out[:B, :C]


# speedup vs baseline: 1.4825x; 1.4825x over previous
import functools

import numpy as np
import jax
import jax.numpy as jnp
from jax.experimental import pallas as pl
from jax.experimental.pallas import tpu as pltpu

_N = 32
_OH1 = 28
_P1 = 14
_OH2 = 10
_D = 5
_F = 16 * _D * _D
_CPAD = 128
_B_TILE = 512
_K1 = 5 * _N


def _conv1_sel_np(parity):
    s = np.zeros((5, 5, _K1, _P1), np.float32)
    for j in range(5):
        for kw in range(5):
            for pw in range(_P1):
                s[j, kw, j * _N + 2 * pw + parity + kw, pw] = 1.0
    return s


def _conv2_sel_np():
    s = np.zeros((25, _P1 * _P1, _OH2 * _OH2), np.float32)
    for kh in range(5):
        for kw in range(5):
            for ph in range(_OH2):
                for pw in range(_OH2):
                    s[kh * 5 + kw, (ph + kh) * _P1 + (pw + kw),
                      ph * _OH2 + pw] = 1.0
    return s.reshape(25, _P1, _P1, _OH2 * _OH2)


def _pool2_mat_np():
    m = np.zeros((16 * _OH2 * _OH2, _F), np.float32)
    for c in range(16):
        for oh in range(_D):
            for ow in range(_D):
                col = c * _D * _D + oh * _D + ow
                for dh in range(2):
                    for dw in range(2):
                        m[c * 100 + (2 * oh + dh) * _OH2 + (2 * ow + dw),
                          col] = 0.25
    return m


_CONV1_SEL_E = _conv1_sel_np(0)
_CONV1_SEL_O = _conv1_sel_np(1)
_CONV2_SEL = _conv2_sel_np()
_POOL2 = _pool2_mat_np()


def _lenet_body(x_ref, w1st_ref, b1_ref, c2_ref, b2_ref, p2_ref,
                wf1_ref, wf2_ref, wf3_ref, bfc_ref,
                o_ref, *, b_tile):
    f32 = jnp.float32
    bf16 = jnp.bfloat16

    xf = x_ref[...]
    xh = xf.astype(bf16)
    xl = (xf - xh.astype(f32)).astype(bf16)

    rows = []
    for r in range(_OH1):
        hs = [xh[:, (r + j) * _N:(r + j + 1) * _N] for j in range(5)]
        ls = [xl[:, (r + j) * _N:(r + j + 1) * _N] for j in range(5)]
        rows.append(jnp.concatenate(hs + ls + hs, axis=1))
    xb = jnp.concatenate(rows, axis=0)

    b1 = b1_ref[...]
    ze = jnp.dot(xb, w1st_ref[:, :6 * _P1], preferred_element_type=f32) + b1
    zo = jnp.dot(xb, w1st_ref[:, 6 * _P1:], preferred_element_type=f32) + b1
    ae = jnp.tanh(ze).astype(bf16).astype(f32).reshape(_OH1, b_tile, 6 * _P1)
    ao = jnp.tanh(zo).astype(bf16).astype(f32).reshape(_OH1, b_tile, 6 * _P1)

    ht = jnp.concatenate(
        [0.25 * (ae[2 * p] + ae[2 * p + 1] + ao[2 * p] + ao[2 * p + 1])
         for p in range(_P1)], axis=1)
    z2 = jnp.dot(ht.astype(bf16), c2_ref[...],
                 preferred_element_type=f32) + b2_ref[...]
    a2 = jnp.tanh(z2)

    feats = jnp.dot(a2.astype(bf16), p2_ref[...],
                    preferred_element_type=f32)

    h = jnp.tanh(jnp.dot(feats.astype(bf16), wf1_ref[...],
                         preferred_element_type=f32) + bfc_ref[:, :120])
    h = jnp.tanh(jnp.dot(h.astype(bf16), wf2_ref[...],
                         preferred_element_type=f32) + bfc_ref[:, 120:204])
    logits = jnp.dot(h.astype(bf16), wf3_ref[...],
                     preferred_element_type=f32) + bfc_ref[:, 204:]
    logits = logits - jnp.max(logits, axis=-1, keepdims=True)
    e = jnp.exp(logits)
    o_ref[...] = e * pl.reciprocal(jnp.sum(e, axis=-1, keepdims=True),
                                   approx=True)


def kernel(x, w_conv1, b_conv1, w_conv2, b_conv2, w_fc1, b_fc1,
           w_fc2, b_fc2, w_fc3, b_fc3, mean, std):
    f32 = jnp.float32
    bf16 = jnp.bfloat16
    B = x.shape[0]
    C = w_fc3.shape[0]
    b_tile = _B_TILE

    w1 = w_conv1.reshape(6, 5, 5).astype(f32)
    w1s = w1 / std
    b1s = b_conv1.astype(f32) - (mean / std) * jnp.sum(w1, axis=(1, 2))
    w1h = w1s.astype(bf16).astype(f32)
    w1l = w1s - w1h
    sele = jnp.asarray(_CONV1_SEL_E)
    selo = jnp.asarray(_CONV1_SEL_O)

    def embed(w):
        return jnp.concatenate(
            [jnp.einsum("cjk,jkrw->rcw", w, sele).reshape(_K1, 6 * _P1),
             jnp.einsum("cjk,jkrw->rcw", w, selo).reshape(_K1, 6 * _P1)],
            axis=1)

    w1h_m = embed(w1h)
    w1l_m = embed(w1l)
    w1st = jnp.concatenate(
        [w1h_m, w1h_m, w1l_m], axis=0).astype(bf16)
    b1row = jnp.repeat(b1s, _P1).reshape(1, 6 * _P1)

    w2r = jnp.transpose(w_conv2.astype(f32), (1, 2, 3, 0)).reshape(6, 25, 16)
    c2big = jnp.einsum("cao,ahwp->hcwop", w2r, jnp.asarray(_CONV2_SEL))
    c2big = c2big.reshape(_P1 * 6 * _P1, 16 * 100).astype(bf16)
    b2row = jnp.repeat(b_conv2.astype(f32), _OH2 * _OH2).reshape(1, 1600)

    pool2 = jnp.asarray(_POOL2, bf16)

    wf1 = w_fc1.T.astype(bf16)
    wf2 = w_fc2.T.astype(bf16)
    wf3p = jnp.zeros((84, _CPAD), f32).at[:, :C].set(
        w_fc3.T.astype(f32)).astype(bf16)
    bf3p = jnp.full((_CPAD,), -1e30, f32).at[:C].set(b_fc3.astype(f32))
    bias_fc = jnp.concatenate(
        [b_fc1.astype(f32), b_fc2.astype(f32), bf3p]).reshape(1, 204 + _CPAD)

    Bp = ((B + b_tile - 1) // b_tile) * b_tile
    x2d = x.reshape(B, _N * _N).astype(f32)
    if Bp != B:
        x2d = jnp.pad(x2d, ((0, Bp - B), (0, 0)))

    weights = (w1st, b1row, c2big, b2row, pool2, wf1, wf2, wf3p, bias_fc)

    def run(xs, w1st, b1row, c2big, b2row, pool2, wf1, wf2, wf3p,
            bias_fc):
        bs = xs.shape[0]
        kern = functools.partial(_lenet_body, b_tile=b_tile)
        const = lambda i: (0, 0)
        return pl.pallas_call(
            kern,
            out_shape=jax.ShapeDtypeStruct((bs, _CPAD), f32),
            grid=(bs // b_tile,),
            in_specs=[
                pl.BlockSpec((b_tile, _N * _N), lambda i: (i, 0)),
                pl.BlockSpec(w1st.shape, const),
                pl.BlockSpec(b1row.shape, const),
                pl.BlockSpec(c2big.shape, const),
                pl.BlockSpec(b2row.shape, const),
                pl.BlockSpec(pool2.shape, const),
                pl.BlockSpec(wf1.shape, const),
                pl.BlockSpec(wf2.shape, const),
                pl.BlockSpec(wf3p.shape, const),
                pl.BlockSpec(bias_fc.shape, const),
            ],
            out_specs=pl.BlockSpec((b_tile, _CPAD), lambda i: (i, 0)),
            compiler_params=pltpu.CompilerParams(
                dimension_semantics=("parallel",)),
        )(xs, w1st, b1row, c2big, b2row, pool2, wf1, wf2, wf3p, bias_fc)

    out = run(x2d, *weights)
    return out[:B, :C]
